# Initial kernel scaffold; baseline (speedup 1.0000x reference)
import jax
import jax.numpy as jnp
from jax.experimental import pallas as pl


def kernel(graph_nodes, graph_edge_links, graph_edges, mask, W1, b1, W2, b2, W3, b3, W4, b4, Wl1, bl1, Wl2, bl2, Wl3, bl3):
    raise NotImplementedError("write your pallas kernel here")


# speedup vs baseline: 1.5258x; 1.5258x over previous
import jax
import jax.numpy as jnp
from jax.experimental import pallas as pl
from jax.experimental.pallas import tpu as pltpu

B, N, E, F_IN, HID, OUT = 4, 10000, 160000, 128, 128, 64
NT = B * N


def _leaky(v):
    return jnp.where(v >= 0, v, 0.01 * v)


def _head_body(x_ref, wl1_ref, bl1_ref, wl2_ref, bl2_ref, wl3_ref, bl3_ref,
               o_ref):
    x = x_ref[...]
    y = _leaky(jnp.dot(x, wl1_ref[...], preferred_element_type=jnp.float32)
               + bl1_ref[...])
    y = _leaky(jnp.dot(y, wl2_ref[...], preferred_element_type=jnp.float32)
               + bl2_ref[...])
    y = jnp.dot(y, wl3_ref[...], preferred_element_type=jnp.float32) + bl3_ref[...]
    o_ref[...] = y


def _softmax_body(s_ref, m_ref, o_ref):
    s = s_ref[...]
    s = jnp.where(m_ref[...] != 0, s, -jnp.inf)
    mx = jnp.max(s, axis=1, keepdims=True)
    e = jnp.exp(s - mx)
    o_ref[...] = e / jnp.sum(e, axis=1, keepdims=True)


def kernel(graph_nodes, graph_edge_links, graph_edges, mask,
           W1, b1, W2, b2, W3, b3, W4, b4, Wl1, bl1, Wl2, bl2, Wl3, bl3):
    x = graph_nodes.reshape(NT, F_IN)
    offsets = (jnp.arange(B, dtype=graph_edge_links.dtype) * N)[:, None, None]
    ei = graph_edge_links + offsets
    ei = jnp.transpose(ei, (1, 0, 2)).reshape(2, B * E)
    row, col = ei[0], ei[1]
    deg = jnp.zeros((NT,), jnp.float32).at[col].add(1.0) + 1.0
    dis = 1.0 / jnp.sqrt(deg)

    def conv(x, W, b):
        h = (x @ W) * dis[:, None]
        agg = jnp.zeros((NT, W.shape[1]), jnp.float32).at[col].add(h[row])
        return _leaky(dis[:, None] * (agg + h) + b)

    x = conv(x, W1, b1)
    x = conv(x, W2, b2)
    x = conv(x, W3, b3)
    x = conv(x, W4, b4)

    nb = 2000
    scores = pl.pallas_call(
        _head_body,
        grid=(NT // nb,),
        in_specs=[
            pl.BlockSpec((nb, HID), lambda i: (i, 0)),
            pl.BlockSpec((HID, HID), lambda i: (0, 0)),
            pl.BlockSpec((HID,), lambda i: (0,)),
            pl.BlockSpec((HID, OUT), lambda i: (0, 0)),
            pl.BlockSpec((OUT,), lambda i: (0,)),
            pl.BlockSpec((OUT, 1), lambda i: (0, 0)),
            pl.BlockSpec((1,), lambda i: (0,)),
        ],
        out_specs=pl.BlockSpec((nb, 1), lambda i: (i, 0)),
        out_shape=jax.ShapeDtypeStruct((NT, 1), jnp.float32),
    )(x, Wl1, bl1, Wl2, bl2, Wl3, bl3)

    s = scores.reshape(B, N)
    return pl.pallas_call(
        _softmax_body,
        out_shape=jax.ShapeDtypeStruct((B, N), jnp.float32),
    )(s, mask)


# speedup vs baseline: 4.6804x; 3.0674x over previous
import jax
import jax.numpy as jnp
from jax import lax
from jax.experimental import pallas as pl
from jax.experimental.pallas import tpu as pltpu
from jax.experimental.pallas import tpu_sc as plsc

B, N, E, F_IN, HID, OUT = 4, 10000, 160000, 128, 128, 64
NT = B * N
NC, NS = 2, 16
FC = 16
NCH = HID // FC
ET = B * E
G = 128
NG = 314
EPT = NG * G
ET_PAD = NS * EPT
NT_ACC = 40064
APT = NT_ACC // NS
ZR = 313
NB = 1000

_f32 = jnp.float32
_sds = jax.ShapeDtypeStruct

_mesh = plsc.VectorSubcoreMesh(
    core_axis_name="c", subcore_axis_name="s", num_cores=NC, num_subcores=NS)


def _agg_body(*refs):
    tables = refs[:NCH]
    row_hbm, col_hbm = refs[NCH], refs[NCH + 1]
    outs = refs[NCH + 2:2 * NCH + 2]
    row_v, col_v, gbuf, z_v, acc, gsem, ssem = refs[2 * NCH + 2:]

    cid = lax.axis_index("c")
    sid = lax.axis_index("s")
    pltpu.sync_copy(row_hbm.at[sid], row_v)
    pltpu.sync_copy(col_hbm.at[sid], col_v)

    @pl.loop(0, ZR)
    def _(i):
        for k in range(FC // 16):
            z_v[i, pl.ds(16 * k, 16)] = jnp.zeros((16,), _f32)

    base = sid * APT
    for c in range(NCH):

        @pl.when(cid == c // (NCH // 2))
        def _(c=c):
            hc = tables[c]
            ac = outs[c]
            for j in range(8):
                pltpu.sync_copy(z_v, acc.at[pl.ds(base + j * ZR, ZR)])
            plsc.subcore_barrier()
            for b in range(2):
                pltpu.async_copy(hc.at[row_v.at[b]], gbuf.at[b], gsem.at[b])

            @pl.loop(0, NG, step=2)
            def _(g):
                for b in range(2):
                    pltpu.make_async_copy(
                        hc.at[row_v.at[g + b]], gbuf.at[b], gsem.at[b]).wait()
                    pltpu.async_copy(
                        gbuf.at[b], acc.at[col_v.at[g + b]], ssem.at[b], add=True)
                for b in range(2):
                    nxt = g + 2 + b

                    @pl.when(nxt < NG)
                    def _(b=b, nxt=nxt, g=g):
                        pltpu.make_async_copy(
                            gbuf.at[b], acc.at[col_v.at[g + b]], ssem.at[b]).wait()
                        pltpu.async_copy(
                            hc.at[row_v.at[nxt]], gbuf.at[b], gsem.at[b])

            for b in range(2):
                pltpu.make_async_copy(
                    gbuf.at[b], acc.at[col_v.at[NG - 2 + b]], ssem.at[b]).wait()
            plsc.subcore_barrier()
            pltpu.sync_copy(acc.at[pl.ds(sid * APT, APT)],
                            ac.at[pl.ds(sid * APT, APT)])
            plsc.subcore_barrier()


_agg_call = pl.kernel(
    _agg_body,
    out_type=[_sds((NT_ACC, FC), _f32)] * NCH,
    mesh=_mesh,
    scratch_types=[
        pltpu.VMEM((NG, G), jnp.int32),
        pltpu.VMEM((NG, G), jnp.int32),
        pltpu.VMEM((2, G, FC), _f32),
        pltpu.VMEM((ZR, FC), _f32),
        pltpu.VMEM_SHARED((NT_ACC, FC), _f32),
        pltpu.SemaphoreType.DMA((2,)),
        pltpu.SemaphoreType.DMA((2,)),
    ],
    compiler_params=pltpu.CompilerParams(use_tc_tiling_on_sc=False),
)


def _leaky(v):
    return jnp.where(v >= 0, v, 0.01 * v)


def _prep_body(*refs):
    hist_ref, x_ref, w_ref, dis_ref = refs[:4]
    h_refs = refs[4:]
    deg = 1.0 + hist_ref[...][:, 0]
    dis = lax.rsqrt(deg)[:, None]
    h = jnp.dot(x_ref[...], w_ref[...], preferred_element_type=_f32) * dis
    dis_ref[...] = dis
    for c, r in enumerate(h_refs):
        r[...] = h[:, c * FC:(c + 1) * FC]


def _layer_body(*refs):
    a_refs = refs[:NCH]
    h_refs = refs[NCH:2 * NCH]
    dis_ref, b_ref, w_ref = refs[2 * NCH:2 * NCH + 3]
    o_refs = refs[2 * NCH + 3:]
    agg = jnp.concatenate([r[...] for r in a_refs], axis=1)
    hh = jnp.concatenate([r[...] for r in h_refs], axis=1)
    dis = dis_ref[...]
    x = _leaky(dis * (agg + hh) + b_ref[...])
    hn = jnp.dot(x, w_ref[...], preferred_element_type=_f32) * dis
    for c, r in enumerate(o_refs):
        r[...] = hn[:, c * FC:(c + 1) * FC]


def _final_body(*refs):
    a_refs = refs[:NCH]
    h_refs = refs[NCH:2 * NCH]
    dis_ref, b_ref, wl1, bl1, wl2, bl2, wl3, bl3, o_ref = refs[2 * NCH:]
    agg = jnp.concatenate([r[...] for r in a_refs], axis=1)
    hh = jnp.concatenate([r[...] for r in h_refs], axis=1)
    dis = dis_ref[...]
    x = _leaky(dis * (agg + hh) + b_ref[...])
    y = _leaky(jnp.dot(x, wl1[...], preferred_element_type=_f32) + bl1[...])
    y = _leaky(jnp.dot(y, wl2[...], preferred_element_type=_f32) + bl2[...])
    o_ref[...] = jnp.dot(y, wl3[...], preferred_element_type=_f32) + bl3[...]


def _softmax_body(s_ref, m_ref, o_ref):
    s = jnp.where(m_ref[...] != 0, s_ref[...], -jnp.inf)
    mx = jnp.max(s, axis=1, keepdims=True)
    e = jnp.exp(s - mx)
    o_ref[...] = e / jnp.sum(e, axis=1, keepdims=True)


def _blk(shape, imap):
    return pl.BlockSpec(shape, imap)


_chunk_spec = _blk((NB, FC), lambda i: (i, 0))
_dis_spec = _blk((NB, 1), lambda i: (i, 0))
_w_spec = _blk((HID, HID), lambda i: (0, 0))
_b_spec = _blk((1, HID), lambda i: (0, 0))

_prep_call = pl.pallas_call(
    _prep_body,
    grid=(NT // NB,),
    in_specs=[_chunk_spec, _blk((NB, F_IN), lambda i: (i, 0)), _w_spec],
    out_specs=[_dis_spec] + [_chunk_spec] * NCH,
    out_shape=[_sds((NT, 1), _f32)] + [_sds((NT, FC), _f32)] * NCH,
)

_layer_call = pl.pallas_call(
    _layer_body,
    grid=(NT // NB,),
    in_specs=[_chunk_spec] * (2 * NCH) + [_dis_spec, _b_spec, _w_spec],
    out_specs=[_chunk_spec] * NCH,
    out_shape=[_sds((NT, FC), _f32)] * NCH,
)

_final_call = pl.pallas_call(
    _final_body,
    grid=(NT // NB,),
    in_specs=[_chunk_spec] * (2 * NCH) + [
        _dis_spec, _b_spec,
        _blk((HID, HID), lambda i: (0, 0)), _blk((1, HID), lambda i: (0, 0)),
        _blk((HID, OUT), lambda i: (0, 0)), _blk((1, OUT), lambda i: (0, 0)),
        _blk((OUT, 1), lambda i: (0, 0)), _blk((1, 1), lambda i: (0, 0)),
    ],
    out_specs=_dis_spec,
    out_shape=_sds((NT, 1), _f32),
)

_softmax_call = pl.pallas_call(
    _softmax_body,
    out_shape=_sds((B, N), _f32),
)


def kernel(graph_nodes, graph_edge_links, graph_edges, mask,
           W1, b1, W2, b2, W3, b3, W4, b4, Wl1, bl1, Wl2, bl2, Wl3, bl3):
    x0 = graph_nodes.reshape(NT, F_IN)
    offsets = (jnp.arange(B, dtype=graph_edge_links.dtype) * N)[:, None, None]
    ei = jnp.transpose(graph_edge_links + offsets, (1, 0, 2)).reshape(2, ET)
    pad_r = jnp.zeros((ET_PAD - ET,), ei.dtype)
    pad_c = jnp.full((ET_PAD - ET,), NT, ei.dtype)
    row_r = jnp.concatenate([ei[0], pad_r]).reshape(NS, NG, G)
    col_r = jnp.concatenate([ei[1], pad_c]).reshape(NS, NG, G)

    ones_t = jnp.ones((NT, FC), _f32)
    hist = _agg_call(*([ones_t] * NCH), row_r, col_r)[0]
    dis, *hs = _prep_call(hist, x0, W1)

    for b_prev, W_next in ((b1, W2), (b2, W3), (b3, W4)):
        aggs = _agg_call(*hs, row_r, col_r)
        hs = _layer_call(*aggs, *hs, dis, b_prev.reshape(1, HID), W_next)

    aggs = _agg_call(*hs, row_r, col_r)
    scores = _final_call(*aggs, *hs, dis, b4.reshape(1, HID),
                         Wl1, bl1.reshape(1, HID), Wl2, bl2.reshape(1, OUT),
                         Wl3, bl3.reshape(1, 1))
    return _softmax_call(scores.reshape(B, N), mask)


# speedup vs baseline: 10.6393x; 2.2732x over previous
import jax
import jax.numpy as jnp
from jax import lax
from jax.experimental import pallas as pl
from jax.experimental.pallas import tpu as pltpu
from jax.experimental.pallas import tpu_sc as plsc

B, N, E, F_IN, HID, OUT = 4, 10000, 160000, 128, 128, 64
NT = B * N
NC, NS = 2, 16
FC = 32
NCH = HID // FC
NPASS = NCH // NC
ET = B * E
G = 128
NG = 314
NGC = 64
NBLK = 5
NG_PAD = NGC * NBLK
NBUF = 8
EPT = NG * G
ET_PAD = NS * NG_PAD * G
NT_ACC = 40064
APT = NT_ACC // NS
ZR = 313
HW = 16
HBUF = 2
NB = 1000

_f32 = jnp.float32
_sds = jax.ShapeDtypeStruct

_mesh = plsc.VectorSubcoreMesh(
    core_axis_name="c", subcore_axis_name="s", num_cores=NC, num_subcores=NS)


def _agg_body(tt_hbm, row4_hbm, col_hbm, out_hbm,
              row_v, col_v, gbuf, acc, gsem, ssem):
    cid = lax.axis_index("c")
    sid = lax.axis_index("s")
    base = sid * APT

    @pl.loop(0, NPASS)
    def _(p):
        chunk = cid * NPASS + p

        @pl.loop(0, G)
        def _(i):
            for b in range(NBUF):
                for k in range(FC // 16):
                    gbuf[b, i, pl.ds(16 * k, 16)] = jnp.zeros((16,), _f32)

        for j in range(19):
            pltpu.sync_copy(gbuf.at[0], acc.at[pl.ds(base + j * G, G)])
        pltpu.sync_copy(gbuf.at[0, pl.ds(0, 72)],
                        acc.at[pl.ds(base + 19 * G, 72)])
        plsc.subcore_barrier()

        @pl.loop(0, NBLK)
        def _(m):
            g_base = m * NGC
            pltpu.sync_copy(row4_hbm.at[chunk, sid, pl.ds(g_base, NGC)], row_v)
            pltpu.sync_copy(col_hbm.at[sid, pl.ds(g_base, NGC)], col_v)
            for b in range(NBUF):

                @pl.when(g_base + b < NG)
                def _(b=b):
                    pltpu.async_copy(tt_hbm.at[row_v.at[b]], gbuf.at[b],
                                     gsem.at[b])

            @pl.loop(0, NGC, step=NBUF)
            def _(g):
                for b in range(NBUF):
                    gl = g + b

                    @pl.when(g_base + gl < NG)
                    def _(b=b, gl=gl):
                        pltpu.make_async_copy(
                            tt_hbm.at[row_v.at[gl]], gbuf.at[b],
                            gsem.at[b]).wait()
                        pltpu.async_copy(
                            gbuf.at[b], acc.at[col_v.at[gl]], ssem.at[b],
                            add=True)
                for b in range(NBUF):
                    gl = g + b
                    nl = g + NBUF + b

                    @pl.when(g_base + gl < NG)
                    def _(b=b, gl=gl):
                        pltpu.make_async_copy(
                            gbuf.at[b], acc.at[col_v.at[gl]],
                            ssem.at[b]).wait()

                    @pl.when(jnp.logical_and(nl < NGC, g_base + nl < NG))
                    def _(b=b, nl=nl):
                        pltpu.async_copy(tt_hbm.at[row_v.at[nl]], gbuf.at[b],
                                         gsem.at[b])

        plsc.subcore_barrier()
        pltpu.sync_copy(acc.at[pl.ds(base, APT)],
                        out_hbm.at[chunk, pl.ds(base, APT)])
        plsc.subcore_barrier()


_agg_call = pl.kernel(
    _agg_body,
    out_type=_sds((NCH, NT_ACC, FC), _f32),
    mesh=_mesh,
    scratch_types=[
        pltpu.VMEM((NGC, G), jnp.int32),
        pltpu.VMEM((NGC, G), jnp.int32),
        pltpu.VMEM((NBUF, G, FC), _f32),
        pltpu.VMEM_SHARED((NT_ACC, FC), _f32),
        pltpu.SemaphoreType.DMA((NBUF,)),
        pltpu.SemaphoreType.DMA((NBUF,)),
    ],
    compiler_params=pltpu.CompilerParams(use_tc_tiling_on_sc=False),
)

_HG = NG // 2 + 1


def _hist_body(col_hbm, ones_hbm, zero_hbm, out_hbm, col_v, ones_v, z_v, acc,
               hsem):
    cid = lax.axis_index("c")
    sid = lax.axis_index("s")
    pltpu.sync_copy(col_hbm.at[sid], col_v)
    pltpu.sync_copy(ones_hbm, ones_v)
    pltpu.sync_copy(zero_hbm, z_v)

    base = sid * APT
    for j in range(8):
        pltpu.sync_copy(z_v, acc.at[pl.ds(base + j * ZR, ZR)])
    plsc.subcore_barrier()

    g0 = cid * _HG

    @pl.loop(0, _HG, step=HBUF)
    def _(g):
        for b in range(HBUF):
            gg = g0 + g + b

            @pl.when(jnp.logical_and(g + b < _HG, gg < NG))
            def _(gg=gg):
                pltpu.async_copy(ones_v, acc.at[col_v.at[gg]], hsem, add=True)
        for b in range(HBUF):
            gg = g0 + g + b

            @pl.when(jnp.logical_and(g + b < _HG, gg < NG))
            def _(gg=gg):
                pltpu.make_async_copy(ones_v, acc.at[col_v.at[gg]],
                                      hsem).wait()

    plsc.subcore_barrier()
    wid = cid * NS + sid
    pltpu.sync_copy(acc.at[pl.ds(base, APT)], out_hbm.at[wid])


_hist_call = pl.kernel(
    _hist_body,
    out_type=_sds((NC * NS, APT, HW), _f32),
    mesh=_mesh,
    scratch_types=[
        pltpu.VMEM((NG_PAD, G), jnp.int32),
        pltpu.VMEM((G, HW), _f32),
        pltpu.VMEM((ZR, HW), _f32),
        pltpu.VMEM_SHARED((NT_ACC, HW), _f32),
        pltpu.SemaphoreType.DMA,
    ],
    compiler_params=pltpu.CompilerParams(use_tc_tiling_on_sc=False),
)


def _leaky(v):
    return jnp.where(v >= 0, v, 0.01 * v)


def _prep_body(hist_ref, x_ref, w_ref, dis_ref, h_ref):
    ph = hist_ref[...]
    deg = 1.0 + ph[0, :, 0] + ph[1, :, 0]
    dis = lax.rsqrt(deg)[:, None]
    h = jnp.dot(x_ref[...], w_ref[...], preferred_element_type=_f32) * dis
    dis_ref[...] = dis
    for c in range(NCH):
        h_ref[c] = h[:, c * FC:(c + 1) * FC]


def _layer_body(a_ref, h_ref, dis_ref, b_ref, w_ref, o_ref):
    agg = jnp.concatenate([a_ref[c] for c in range(NCH)], axis=1)
    hh = jnp.concatenate([h_ref[c] for c in range(NCH)], axis=1)
    dis = dis_ref[...]
    x = _leaky(dis * (agg + hh) + b_ref[...])
    hn = jnp.dot(x, w_ref[...], preferred_element_type=_f32) * dis
    for c in range(NCH):
        o_ref[c] = hn[:, c * FC:(c + 1) * FC]


def _final_body(a_ref, h_ref, dis_ref, b_ref, wl1, bl1, wl2, bl2, wl3, bl3,
                o_ref):
    agg = jnp.concatenate([a_ref[c] for c in range(NCH)], axis=1)
    hh = jnp.concatenate([h_ref[c] for c in range(NCH)], axis=1)
    dis = dis_ref[...]
    x = _leaky(dis * (agg + hh) + b_ref[...])
    y = _leaky(jnp.dot(x, wl1[...], preferred_element_type=_f32) + bl1[...])
    y = _leaky(jnp.dot(y, wl2[...], preferred_element_type=_f32) + bl2[...])
    o_ref[...] = jnp.dot(y, wl3[...], preferred_element_type=_f32) + bl3[...]


def _softmax_body(s_ref, m_ref, o_ref):
    s = jnp.where(m_ref[...] != 0, s_ref[...], -jnp.inf)
    mx = jnp.max(s, axis=1, keepdims=True)
    e = jnp.exp(s - mx)
    o_ref[...] = e / jnp.sum(e, axis=1, keepdims=True)


def _blk(shape, imap):
    return pl.BlockSpec(shape, imap)


_stack_spec = _blk((NCH, NB, FC), lambda i: (0, i, 0))
_dis_spec = _blk((NB, 1), lambda i: (i, 0))
_w_spec = _blk((HID, HID), lambda i: (0, 0))
_b_spec = _blk((1, HID), lambda i: (0, 0))

_prep_call = pl.pallas_call(
    _prep_body,
    grid=(NT // NB,),
    in_specs=[_blk((NC, NB, HW), lambda i: (0, i, 0)),
              _blk((NB, F_IN), lambda i: (i, 0)), _w_spec],
    out_specs=[_dis_spec, _stack_spec],
    out_shape=[_sds((NT, 1), _f32), _sds((NCH, NT, FC), _f32)],
)

_layer_call = pl.pallas_call(
    _layer_body,
    grid=(NT // NB,),
    in_specs=[_stack_spec, _stack_spec, _dis_spec, _b_spec, _w_spec],
    out_specs=_stack_spec,
    out_shape=_sds((NCH, NT, FC), _f32),
)

_final_call = pl.pallas_call(
    _final_body,
    grid=(NT // NB,),
    in_specs=[_stack_spec, _stack_spec, _dis_spec, _b_spec,
              _blk((HID, HID), lambda i: (0, 0)),
              _blk((1, HID), lambda i: (0, 0)),
              _blk((HID, OUT), lambda i: (0, 0)),
              _blk((1, OUT), lambda i: (0, 0)),
              _blk((OUT, 1), lambda i: (0, 0)),
              _blk((1, 1), lambda i: (0, 0))],
    out_specs=_dis_spec,
    out_shape=_sds((NT, 1), _f32),
)

_softmax_call = pl.pallas_call(
    _softmax_body,
    out_shape=_sds((B, N), _f32),
)


def kernel(graph_nodes, graph_edge_links, graph_edges, mask,
           W1, b1, W2, b2, W3, b3, W4, b4, Wl1, bl1, Wl2, bl2, Wl3, bl3):
    x0 = graph_nodes.reshape(NT, F_IN)
    offsets = (jnp.arange(B, dtype=graph_edge_links.dtype) * N)[:, None, None]
    ei = jnp.transpose(graph_edge_links + offsets, (1, 0, 2)).reshape(2, ET)
    ep1 = NS * NG * G
    tpad = (NG_PAD - NG) * G

    def _pack(v, fill):
        v = jnp.concatenate([v, jnp.full((ep1 - ET,), fill, v.dtype)])
        v = v.reshape(NS, NG * G)
        v = jnp.pad(v, ((0, 0), (0, tpad)), constant_values=fill)
        return v.reshape(NS, NG_PAD, G)

    row_r = _pack(ei[0], 0)
    col_r = _pack(ei[1], NT)
    row4 = row_r[None] + (jnp.arange(NCH, dtype=ei.dtype) * NT)[:, None, None,
                                                                None]

    ones_sc = jnp.ones((G, HW), _f32)
    zero_sc = jnp.zeros((ZR, HW), _f32)
    hist = _hist_call(col_r, ones_sc, zero_sc).reshape(NC, NT_ACC, HW)
    dis, h = _prep_call(hist, x0, W1)

    for b_prev, W_next in ((b1, W2), (b2, W3), (b3, W4)):
        agg = _agg_call(h.reshape(NCH * NT, FC), row4, col_r)
        h = _layer_call(agg, h, dis, b_prev.reshape(1, HID), W_next)

    agg = _agg_call(h.reshape(NCH * NT, FC), row4, col_r)
    scores = _final_call(agg, h, dis, b4.reshape(1, HID),
                         Wl1, bl1.reshape(1, HID), Wl2, bl2.reshape(1, OUT),
                         Wl3, bl3.reshape(1, 1))
    return _softmax_call(scores.reshape(B, N), mask)
